# baseline (speedup 1.0000x reference)
import functools

import jax
import jax.numpy as jnp
from jax import lax
from jax.experimental import pallas as pl
from jax.experimental.pallas import tpu as pltpu
from jax.experimental.pallas import tpu_sc as plsc

_NC = 2
_NS = 16
_NW = _NC * _NS
_LANES = 16
_EMB = 64
_BATCH = 16384
_BPW = _BATCH // _NW
_CHUNK = 128
_NCHUNK = _BPW // _CHUNK
_GROUPS = _BPW // _LANES


def _mf_body(u2, i2, ue, ub1, ie, ib1, mean16, out,
             uidx, iidx, urows, irows, ubias, ibias, outv, meanv, sem):
    wid = lax.axis_index("s") * _NC + lax.axis_index("c")
    base = wid * _BPW

    pltpu.sync_copy(u2.at[pl.ds(wid * _NCHUNK, _NCHUNK)], uidx)
    pltpu.sync_copy(i2.at[pl.ds(wid * _NCHUNK, _NCHUNK)], iidx)
    pltpu.sync_copy(mean16, meanv)

    descs = []
    for k in range(_NCHUNK):
        sl = pl.ds(k * _CHUNK, _CHUNK)
        descs.append(pltpu.async_copy(ue.at[uidx.at[k]], urows.at[sl], sem))
        descs.append(pltpu.async_copy(ie.at[iidx.at[k]], irows.at[sl], sem))
        descs.append(pltpu.async_copy(ub1.at[uidx.at[k]], ubias.at[sl], sem))
        descs.append(pltpu.async_copy(ib1.at[iidx.at[k]], ibias.at[sl], sem))
    for d in descs:
        d.wait()

    mean_v = meanv[...]
    lane = lax.iota(jnp.int32, _LANES)
    perms = [lane ^ sh for sh in (8, 4, 2, 1)]

    dnums = lax.GatherDimensionNumbers(
        offset_dims=(), collapsed_slice_dims=(0,), start_index_map=(0,))

    def hsum(v):
        for p in perms:
            v = v + lax.gather(v, p.reshape(_LANES, 1), dnums, (1,),
                               mode=lax.GatherScatterMode.PROMISE_IN_BOUNDS)
        return v

    def body(g, carry):
        j0 = g * _LANES
        sums = jnp.zeros((_LANES,), jnp.float32)
        for jj in range(_LANES):
            j = j0 + jj
            acc = urows[j, pl.ds(0, _LANES)] * irows[j, pl.ds(0, _LANES)]
            for t in range(1, _EMB // _LANES):
                acc = acc + (urows[j, pl.ds(t * _LANES, _LANES)]
                             * irows[j, pl.ds(t * _LANES, _LANES)])
            sums = jnp.where(lane == jj, hsum(acc), sums)
        outv[pl.ds(j0, _LANES)] = (sums + ubias[pl.ds(j0, _LANES)]
                                   + ibias[pl.ds(j0, _LANES)] + mean_v)
        return carry

    lax.fori_loop(0, _GROUPS, body, 0)

    pltpu.sync_copy(outv, out.at[pl.ds(base, _BPW)])


@jax.jit
def _mf(u2, i2, ue, ub1, ie, ib1, mean16):
    mesh = plsc.VectorSubcoreMesh(core_axis_name="c", subcore_axis_name="s")
    f = functools.partial(
        pl.kernel,
        out_type=jax.ShapeDtypeStruct((_BATCH,), jnp.float32),
        mesh=mesh,
        compiler_params=pltpu.CompilerParams(use_tc_tiling_on_sc=False),
        scratch_types=[
            pltpu.VMEM((_NCHUNK, _CHUNK), jnp.int32),
            pltpu.VMEM((_NCHUNK, _CHUNK), jnp.int32),
            pltpu.VMEM((_BPW, _EMB), jnp.float32),
            pltpu.VMEM((_BPW, _EMB), jnp.float32),
            pltpu.VMEM((_BPW,), jnp.float32),
            pltpu.VMEM((_BPW,), jnp.float32),
            pltpu.VMEM((_BPW,), jnp.float32),
            pltpu.VMEM((_LANES,), jnp.float32),
            pltpu.SemaphoreType.DMA,
        ],
    )(_mf_body)
    return f(u2, i2, ue, ub1, ie, ib1, mean16)


def kernel(u_id, i_id, user_emb, user_bias, item_emb, item_bias, mean):
    u2 = u_id.astype(jnp.int32).reshape(_NW * _NCHUNK, _CHUNK)
    i2 = i_id.astype(jnp.int32).reshape(_NW * _NCHUNK, _CHUNK)
    mean16 = jnp.broadcast_to(mean.astype(jnp.float32), (_LANES,))
    return _mf(u2, i2, user_emb, user_bias.reshape(-1),
               item_emb, item_bias.reshape(-1), mean16)


# speedup vs baseline: 1.0004x; 1.0004x over previous
import functools

import jax
import jax.numpy as jnp
from jax import lax
from jax.experimental import pallas as pl
from jax.experimental.pallas import tpu as pltpu
from jax.experimental.pallas import tpu_sc as plsc

_NC = 2
_NS = 16
_NW = _NC * _NS
_LANES = 16
_EMB = 64
_BATCH = 16384
_BPW = _BATCH // _NW
_CHUNK = 128
_NCHUNK = _BPW // _CHUNK
_GROUPS = _BPW // _LANES


def _mf_body(u2, i2, ue, ub1, ie, ib1, mean16, out,
             uidx, iidx, urows, irows, ubias, ibias, outv, meanv, sem):
    wid = lax.axis_index("s") * _NC + lax.axis_index("c")
    base = wid * _BPW

    pltpu.sync_copy(u2.at[pl.ds(wid * _NCHUNK, _NCHUNK)], uidx)
    pltpu.sync_copy(i2.at[pl.ds(wid * _NCHUNK, _NCHUNK)], iidx)
    pltpu.sync_copy(mean16, meanv)

    descs = []
    for k in range(_NCHUNK):
        sl = pl.ds(k * _CHUNK, _CHUNK)
        descs.append(pltpu.async_copy(ue.at[uidx.at[k]], urows.at[sl], sem))
        descs.append(pltpu.async_copy(ie.at[iidx.at[k]], irows.at[sl], sem))
        descs.append(pltpu.async_copy(ub1.at[uidx.at[k]], ubias.at[sl], sem))
        descs.append(pltpu.async_copy(ib1.at[iidx.at[k]], ibias.at[sl], sem))
    for d in descs:
        d.wait()

    mean_v = meanv[...]
    lane = lax.iota(jnp.int32, _LANES)
    perms = [lane ^ sh for sh in (8, 4, 2, 1)]

    dnums = lax.GatherDimensionNumbers(
        offset_dims=(), collapsed_slice_dims=(0,), start_index_map=(0,))

    def hsum(v):
        for p in perms:
            v = v + lax.gather(v, p.reshape(_LANES, 1), dnums, (1,),
                               mode=lax.GatherScatterMode.PROMISE_IN_BOUNDS)
        return v

    def body(g, carry):
        j0 = g * _LANES
        sums = jnp.zeros((_LANES,), jnp.float32)
        for jj in range(_LANES):
            j = j0 + jj
            acc = urows[j, pl.ds(0, _LANES)] * irows[j, pl.ds(0, _LANES)]
            for t in range(1, _EMB // _LANES):
                acc = acc + (urows[j, pl.ds(t * _LANES, _LANES)]
                             * irows[j, pl.ds(t * _LANES, _LANES)])
            sums = jnp.where(lane == jj, hsum(acc), sums)
        bu = ubias[pl.ds(j0, _LANES)]
        bi = ibias[pl.ds(j0, _LANES)]
        outv[pl.ds(j0, _LANES)] = sums + bu + bi + mean_v
        return carry

    lax.fori_loop(0, _GROUPS, body, 0)

    pltpu.sync_copy(outv, out.at[pl.ds(base, _BPW)])


@jax.jit
def _mf(u2, i2, ue, ub1, ie, ib1, mean16):
    mesh = plsc.VectorSubcoreMesh(core_axis_name="c", subcore_axis_name="s")
    f = functools.partial(
        pl.kernel,
        out_type=jax.ShapeDtypeStruct((_BATCH,), jnp.float32),
        mesh=mesh,
        compiler_params=pltpu.CompilerParams(use_tc_tiling_on_sc=False),
        scratch_types=[
            pltpu.VMEM((_NCHUNK, _CHUNK), jnp.int32),
            pltpu.VMEM((_NCHUNK, _CHUNK), jnp.int32),
            pltpu.VMEM((_BPW, _EMB), jnp.float32),
            pltpu.VMEM((_BPW, _EMB), jnp.float32),
            pltpu.VMEM((_BPW,), jnp.float32),
            pltpu.VMEM((_BPW,), jnp.float32),
            pltpu.VMEM((_BPW,), jnp.float32),
            pltpu.VMEM((_LANES,), jnp.float32),
            pltpu.SemaphoreType.DMA,
        ],
    )(_mf_body)
    return f(u2, i2, ue, ub1, ie, ib1, mean16)


def kernel(u_id, i_id, user_emb, user_bias, item_emb, item_bias, mean):
    u2 = u_id.astype(jnp.int32).reshape(_NW * _NCHUNK, _CHUNK)
    i2 = i_id.astype(jnp.int32).reshape(_NW * _NCHUNK, _CHUNK)
    mean16 = jnp.broadcast_to(mean.astype(jnp.float32), (_LANES,))
    return _mf(u2, i2, user_emb, user_bias.reshape(-1),
               item_emb, item_bias.reshape(-1), mean16)
